# baseline (speedup 1.0000x reference)
import functools
import math

import jax
import jax.numpy as jnp
from jax import lax
from jax.experimental import pallas as pl
from jax.experimental.pallas import tpu as pltpu
from jax.experimental.pallas import tpu_sc as plsc

NG = 64
DELTA = 20.0 / (NG - 1)
COEFF = -0.5 / (DELTA * DELTA)
C2 = COEFF * math.log2(math.e)
LANES = 16
NW = 32
CHUNK = 400
TPITCH = NG + 1
OPITCH = 2 * NG + 1
NROWS = 100


@functools.lru_cache(maxsize=None)
def _build(E):
    per_w = E // NW
    n_chunks = per_w // CHUNK
    mesh = plsc.VectorSubcoreMesh(
        core_axis_name="c", subcore_axis_name="s", num_cores=2, num_subcores=16
    )

    @functools.partial(
        pl.kernel,
        out_type=jax.ShapeDtypeStruct((E, 2 * NG), jnp.float32),
        mesh=mesh,
        compiler_params=pltpu.CompilerParams(
            use_tc_tiling_on_sc=False, needs_layout_passes=False
        ),
        scratch_types=[
            pltpu.VMEM((per_w,), jnp.float32),
            pltpu.VMEM((per_w,), jnp.int32),
            pltpu.VMEM((NROWS * TPITCH,), jnp.float32),
            pltpu.VMEM((CHUNK, OPITCH), jnp.float32),
        ],
    )
    def sc_kernel(len_hbm, idx_hbm, table_hbm, out_hbm,
                  len_v, idx_v, table_v, out_v):
        wid = lax.axis_index("s") * 2 + lax.axis_index("c")
        lane = lax.iota(jnp.int32, LANES)

        pltpu.sync_copy(len_hbm.at[pl.ds(wid * per_w, per_w)], len_v)
        pltpu.sync_copy(idx_hbm.at[pl.ds(wid * per_w, per_w)], idx_v)
        pltpu.sync_copy(table_hbm, table_v)

        def chunk_body(c, carry):
            base = wid * per_w + c * CHUNK

            def e_body(e, carry2):
                off = c * CHUNK + e * LANES
                d16 = len_v[pl.ds(off, LANES)]
                a16 = idx_v[pl.ds(off, LANES)] * TPITCH
                row = lane + e * LANES
                for g in range(NG):
                    ev = plsc.load_gather(table_v, [a16 + g])
                    plsc.store_scatter(
                        out_v, [row, jnp.full((LANES,), NG + g, jnp.int32)], ev)
                for g in range(NG):
                    t = d16 - (g * DELTA)
                    v = jnp.exp(COEFF * (t * t))
                    plsc.store_scatter(
                        out_v, [row, jnp.full((LANES,), g, jnp.int32)], v)
                return carry2

            lax.fori_loop(0, CHUNK // LANES, e_body, 0, unroll=False)
            pltpu.sync_copy(out_v.at[:, pl.ds(0, 2 * NG)],
                            out_hbm.at[pl.ds(base, CHUNK)])
            return carry

        lax.fori_loop(0, n_chunks, chunk_body, 0, unroll=False)

    return sc_kernel


def kernel(edge_length, edge_type, bond_emb_weight):
    E = edge_length.shape[0]
    lengths = edge_length.reshape(E)
    idx = edge_type.astype(jnp.int32)
    table = jnp.concatenate(
        [bond_emb_weight,
         jnp.zeros((bond_emb_weight.shape[0], TPITCH - NG), jnp.float32)],
        axis=1).reshape(-1)
    fn = _build(E)
    return fn(lengths, idx, table)


# speedup vs baseline: 1.0015x; 1.0015x over previous
import functools
import math

import jax
import jax.numpy as jnp
from jax import lax
from jax.experimental import pallas as pl
from jax.experimental.pallas import tpu as pltpu
from jax.experimental.pallas import tpu_sc as plsc

NG = 64
DELTA = 20.0 / (NG - 1)
COEFF = -0.5 / (DELTA * DELTA)
C2 = COEFF * math.log2(math.e)
LANES = 16
NW = 32
CHUNK = 400
TPITCH = NG + 1
OPITCH = 2 * NG + 1
NROWS = 100


@functools.lru_cache(maxsize=None)
def _build(E):
    per_w = E // NW
    n_chunks = per_w // CHUNK
    mesh = plsc.VectorSubcoreMesh(
        core_axis_name="c", subcore_axis_name="s", num_cores=2, num_subcores=16
    )

    @functools.partial(
        pl.kernel,
        out_type=jax.ShapeDtypeStruct((E, 2 * NG), jnp.float32),
        mesh=mesh,
        compiler_params=pltpu.CompilerParams(
            use_tc_tiling_on_sc=False, needs_layout_passes=False
        ),
        scratch_types=[
            pltpu.VMEM((per_w,), jnp.float32),
            pltpu.VMEM((per_w,), jnp.int32),
            pltpu.VMEM((NROWS * TPITCH,), jnp.float32),
            pltpu.VMEM((CHUNK, OPITCH), jnp.float32),
        ],
    )
    def sc_kernel(len_hbm, idx_hbm, table_hbm, out_hbm,
                  len_v, idx_v, table_v, out_v):
        wid = lax.axis_index("s") * 2 + lax.axis_index("c")
        lane = lax.iota(jnp.int32, LANES)

        pltpu.sync_copy(len_hbm.at[pl.ds(wid * per_w, per_w)], len_v)
        pltpu.sync_copy(idx_hbm.at[pl.ds(wid * per_w, per_w)], idx_v)
        pltpu.sync_copy(table_hbm, table_v)

        def chunk_body(c, carry):
            base = wid * per_w + c * CHUNK

            def e_body(e, carry2):
                off = c * CHUNK + e * LANES
                d16 = len_v[pl.ds(off, LANES)]
                a16 = idx_v[pl.ds(off, LANES)] * TPITCH
                row = lane + e * LANES
                for g in range(NG):
                    t = d16 - (g * DELTA)
                    v = jnp.exp(COEFF * (t * t))
                    plsc.store_scatter(
                        out_v, [row, jnp.full((LANES,), g, jnp.int32)], v)
                    ev = plsc.load_gather(table_v, [a16 + g])
                    plsc.store_scatter(
                        out_v, [row, jnp.full((LANES,), NG + g, jnp.int32)], ev)
                return carry2

            lax.fori_loop(0, CHUNK // LANES, e_body, 0, unroll=2)
            pltpu.sync_copy(out_v.at[:, pl.ds(0, 2 * NG)],
                            out_hbm.at[pl.ds(base, CHUNK)])
            return carry

        lax.fori_loop(0, n_chunks, chunk_body, 0, unroll=False)

    return sc_kernel


def kernel(edge_length, edge_type, bond_emb_weight):
    E = edge_length.shape[0]
    lengths = edge_length.reshape(E)
    idx = edge_type.astype(jnp.int32)
    table = jnp.concatenate(
        [bond_emb_weight,
         jnp.zeros((bond_emb_weight.shape[0], TPITCH - NG), jnp.float32)],
        axis=1).reshape(-1)
    fn = _build(E)
    return fn(lengths, idx, table)


# speedup vs baseline: 1.1020x; 1.1003x over previous
import functools
import math

import jax
import jax.numpy as jnp
from jax import lax
from jax.experimental import pallas as pl
from jax.experimental.pallas import tpu as pltpu
from jax.experimental.pallas import tpu_sc as plsc

NG = 64
DELTA = 20.0 / (NG - 1)
COEFF = -0.5 / (DELTA * DELTA)
C2 = COEFF * math.log2(math.e)
LANES = 16
NW = 32
CHUNK = 400
TPITCH = NG + 1
OPITCH = 2 * NG + 1
NROWS = 100


@functools.lru_cache(maxsize=None)
def _build(E):
    per_w = E // NW
    n_chunks = per_w // CHUNK
    mesh = plsc.VectorSubcoreMesh(
        core_axis_name="c", subcore_axis_name="s", num_cores=2, num_subcores=16
    )

    @functools.partial(
        pl.kernel,
        out_type=jax.ShapeDtypeStruct((E, 2 * NG), jnp.float32),
        mesh=mesh,
        compiler_params=pltpu.CompilerParams(
            use_tc_tiling_on_sc=False, needs_layout_passes=False
        ),
        scratch_types=[
            pltpu.VMEM((per_w,), jnp.float32),
            pltpu.VMEM((per_w,), jnp.int32),
            pltpu.VMEM((NROWS * TPITCH,), jnp.float32),
            pltpu.VMEM((CHUNK, OPITCH), jnp.float32),
        ],
    )
    def sc_kernel(len_hbm, idx_hbm, table_hbm, out_hbm,
                  len_v, idx_v, table_v, out_v):
        wid = lax.axis_index("s") * 2 + lax.axis_index("c")
        lane = lax.iota(jnp.int32, LANES)

        pltpu.sync_copy(len_hbm.at[pl.ds(wid * per_w, per_w)], len_v)
        pltpu.sync_copy(idx_hbm.at[pl.ds(wid * per_w, per_w)], idx_v)
        pltpu.sync_copy(table_hbm, table_v)

        def chunk_body(c, carry):
            base = wid * per_w + c * CHUNK

            def e_body(e, carry2):
                off = c * CHUNK + e * LANES
                d16 = len_v[pl.ds(off, LANES)]
                a16 = idx_v[pl.ds(off, LANES)] * TPITCH
                row = lane + e * LANES
                for g in range(NG):
                    t = d16 - (g * DELTA)
                    v = jnp.exp(COEFF * (t * t))
                    plsc.store_scatter(
                        out_v, [row, jnp.full((LANES,), g, jnp.int32)], v)
                    ev = plsc.load_gather(table_v, [a16 + g])
                    plsc.store_scatter(
                        out_v, [row, jnp.full((LANES,), NG + g, jnp.int32)], ev)
                return carry2

            lax.fori_loop(0, CHUNK // LANES, e_body, 0, unroll=False)
            pltpu.sync_copy(out_v.at[:, pl.ds(0, 2 * NG)],
                            out_hbm.at[pl.ds(base, CHUNK)])
            return carry

        lax.fori_loop(0, n_chunks, chunk_body, 0, unroll=False)

    return sc_kernel


def kernel(edge_length, edge_type, bond_emb_weight):
    E = edge_length.shape[0]
    lengths = edge_length.reshape(E)
    idx = edge_type.astype(jnp.int32)
    table = jnp.concatenate(
        [bond_emb_weight,
         jnp.zeros((bond_emb_weight.shape[0], TPITCH - NG), jnp.float32)],
        axis=1).reshape(-1)
    fn = _build(E)
    return fn(lengths, idx, table)
